# baseline (speedup 1.0000x reference)
import functools

import jax
import jax.numpy as jnp
from jax import lax
from jax.experimental import pallas as pl
from jax.experimental.pallas import tpu as pltpu
from jax.experimental.pallas import tpu_sc as plsc

_NUM_CORES = 2
_NUM_SUBCORES = 16
_BLK_EDGES = 128


def _round_up(a, m):
    return (a + m - 1) // m * m


def _dense_two(h, W_rel, W_root, b, blk_rows):
    R, D = h.shape

    def body(h_ref, wr_ref, wo_ref, b_ref, m_ref, r_ref):
        hb = h_ref[...]
        dn = (((1,), (1,)), ((), ()))
        m_ref[...] = lax.dot_general(hb, wr_ref[...], dn,
                                     preferred_element_type=jnp.float32)
        r_ref[...] = lax.dot_general(hb, wo_ref[...], dn,
                                     preferred_element_type=jnp.float32) + b_ref[...]

    return pl.pallas_call(
        body,
        grid=(R // blk_rows,),
        in_specs=[
            pl.BlockSpec((blk_rows, D), lambda i: (i, 0)),
            pl.BlockSpec((D, D), lambda i: (0, 0)),
            pl.BlockSpec((D, D), lambda i: (0, 0)),
            pl.BlockSpec((1, D), lambda i: (0, 0)),
        ],
        out_specs=[
            pl.BlockSpec((blk_rows, D), lambda i: (i, 0)),
            pl.BlockSpec((blk_rows, D), lambda i: (i, 0)),
        ],
        out_shape=[
            jax.ShapeDtypeStruct((R, D), jnp.float32),
            jax.ShapeDtypeStruct((R, D), jnp.float32),
        ],
    )(h, W_rel, W_root, b)


def _fused_dense_two(parts, r_prev, W_rel, W_root, b, blk_rows):
    _, R, D = parts.shape

    def body(p_ref, rp_ref, wr_ref, wo_ref, b_ref, m_ref, r_ref):
        hb = jnp.maximum(p_ref[0] + p_ref[1] + rp_ref[...], 0.0)
        dn = (((1,), (1,)), ((), ()))
        m_ref[...] = lax.dot_general(hb, wr_ref[...], dn,
                                     preferred_element_type=jnp.float32)
        r_ref[...] = lax.dot_general(hb, wo_ref[...], dn,
                                     preferred_element_type=jnp.float32) + b_ref[...]

    return pl.pallas_call(
        body,
        grid=(R // blk_rows,),
        in_specs=[
            pl.BlockSpec((2, blk_rows, D), lambda i: (0, i, 0)),
            pl.BlockSpec((blk_rows, D), lambda i: (i, 0)),
            pl.BlockSpec((D, D), lambda i: (0, 0)),
            pl.BlockSpec((D, D), lambda i: (0, 0)),
            pl.BlockSpec((1, D), lambda i: (0, 0)),
        ],
        out_specs=[
            pl.BlockSpec((blk_rows, D), lambda i: (i, 0)),
            pl.BlockSpec((blk_rows, D), lambda i: (i, 0)),
        ],
        out_shape=[
            jax.ShapeDtypeStruct((R, D), jnp.float32),
            jax.ShapeDtypeStruct((R, D), jnp.float32),
        ],
    )(parts, r_prev, W_rel, W_root, b)


def _combine(parts, r, blk_rows):
    _, R, D = parts.shape

    def body(p_ref, r_ref, o_ref):
        o_ref[...] = jnp.maximum(p_ref[0] + p_ref[1] + r_ref[...], 0.0)

    return pl.pallas_call(
        body,
        grid=(R // blk_rows,),
        in_specs=[
            pl.BlockSpec((2, blk_rows, D), lambda i: (0, i, 0)),
            pl.BlockSpec((blk_rows, D), lambda i: (i, 0)),
        ],
        out_specs=pl.BlockSpec((blk_rows, D), lambda i: (i, 0)),
        out_shape=jax.ShapeDtypeStruct((R, D), jnp.float32),
    )(parts, r)


def _sc_segsum(m, sdb, zeros, n_acc, rows_per_tile, blocks_per_tile):
    D = m.shape[1]
    bpt = blocks_per_tile
    assert bpt % 4 == 0 and bpt >= 8
    mesh = plsc.VectorSubcoreMesh(core_axis_name="c", subcore_axis_name="s",
                                  num_cores=_NUM_CORES,
                                  num_subcores=_NUM_SUBCORES)

    @functools.partial(
        pl.kernel,
        out_type=jax.ShapeDtypeStruct((_NUM_CORES, n_acc, D), jnp.float32),
        mesh=mesh,
        scratch_types=[
            pltpu.VMEM((4, 2, _BLK_EDGES), jnp.int32),
            pltpu.VMEM((2, _BLK_EDGES, D), jnp.float32),
            pltpu.VMEM_SHARED((n_acc, D), jnp.float32),
        ] + [pltpu.SemaphoreType.DMA] * 3,
    )
    def k(m_hbm, sdb_hbm, z_hbm, out_hbm, idx, rows, acc_sh, gs0, gs1, isem):
        gsem = (gs0, gs1)
        c = lax.axis_index("c")
        s = lax.axis_index("s")
        my_rows = pl.ds(s * rows_per_tile, rows_per_tile)
        base = (c * _NUM_SUBCORES + s) * bpt
        pltpu.sync_copy(z_hbm, acc_sh.at[my_rows])
        plsc.subcore_barrier()

        def ipf(t, q):
            pltpu.async_copy(sdb_hbm.at[base + t], idx.at[q], isem)

        def ipf_wait():
            pltpu.make_async_copy(sdb_hbm.at[base], idx.at[0], isem).wait()

        def gather_start(q, b):
            pltpu.async_copy(m_hbm.at[idx.at[q].at[0]], rows.at[b], gsem[b])

        def gather_wait(b):
            pltpu.make_async_copy(m_hbm.at[idx.at[0].at[0]], rows.at[b],
                                  gsem[b]).wait()

        def scat(q, b):
            pltpu.sync_copy(rows.at[b], acc_sh.at[idx.at[q].at[1]], add=True)

        def step(t, o, gather_next, prefetch):
            b = o % 2
            if gather_next:
                ipf_wait()
                gather_start((o + 1) % 4, 1 - b)
            if prefetch:
                ipf(t + 2, (o + 2) % 4)
            gather_wait(b)
            scat(o, b)

        pltpu.sync_copy(sdb_hbm.at[base], idx.at[0])
        ipf(1, 1)
        gather_start(0, 0)

        @pl.loop(0, (bpt - 4) // 4)
        def _(i):
            t0 = i * 4
            for o in range(4):
                step(t0 + o, o, True, True)

        step(bpt - 4, 0, True, True)
        step(bpt - 3, 1, True, True)
        step(bpt - 2, 2, True, False)
        step(bpt - 1, 3, False, False)

        plsc.subcore_barrier()
        pltpu.sync_copy(acc_sh.at[my_rows], out_hbm.at[c].at[my_rows])

    return k(m, sdb, zeros)


def kernel(x, edge_index, W1_rel, W1_root, b1, W2_rel, W2_root, b2):
    N, D = x.shape
    E = edge_index.shape[1]
    nw = _NUM_CORES * _NUM_SUBCORES

    blocks_per_tile = _round_up(-(-E // (nw * _BLK_EDGES)), 8)
    e_pad = nw * _BLK_EDGES * blocks_per_tile
    rows_per_tile = _round_up(-(-(N + 1) // _NUM_SUBCORES), 64)
    n_acc = _NUM_SUBCORES * rows_per_tile

    src = edge_index[0].astype(jnp.int32)
    dst = edge_index[1].astype(jnp.int32)
    pad = e_pad - E
    srcb = jnp.pad(src, (0, pad), constant_values=N).reshape(e_pad // _BLK_EDGES,
                                                             _BLK_EDGES)
    dstb = jnp.pad(dst, (0, pad), constant_values=N).reshape(e_pad // _BLK_EDGES,
                                                             _BLK_EDGES)
    sdb = jnp.stack([srcb, dstb], axis=1)
    xp = jnp.pad(x, ((0, n_acc - N), (0, 0)))
    zeros = jnp.zeros((rows_per_tile, D), jnp.float32)
    b1r = b1.reshape(1, D)
    b2r = b2.reshape(1, D)

    blk_rows = 1024 if n_acc % 1024 == 0 else 64

    m1, r1 = _dense_two(xp, W1_rel, W1_root, b1r, blk_rows)
    parts1 = _sc_segsum(m1, sdb, zeros, n_acc, rows_per_tile, blocks_per_tile)
    m2, r2 = _fused_dense_two(parts1, r1, W2_rel, W2_root, b2r, blk_rows)
    parts2 = _sc_segsum(m2, sdb, zeros, n_acc, rows_per_tile, blocks_per_tile)
    out = _combine(parts2, r2, blk_rows)
    return out[:N]


# speedup vs baseline: 1.2554x; 1.2554x over previous
import functools

import jax
import jax.numpy as jnp
from jax import lax
from jax.experimental import pallas as pl
from jax.experimental.pallas import tpu as pltpu
from jax.experimental.pallas import tpu_sc as plsc

_NUM_CORES = 2
_NUM_SUBCORES = 16
_BLK_EDGES = 128


def _round_up(a, m):
    return (a + m - 1) // m * m


def _dense_two(h, W_rel, W_root, b, blk_rows):
    R, D = h.shape

    def body(h_ref, wr_ref, wo_ref, b_ref, m_ref, r_ref):
        hb = h_ref[...]
        dn = (((1,), (1,)), ((), ()))
        m_ref[...] = lax.dot_general(hb, wr_ref[...], dn,
                                     preferred_element_type=jnp.float32)
        r_ref[...] = lax.dot_general(hb, wo_ref[...], dn,
                                     preferred_element_type=jnp.float32) + b_ref[...]

    return pl.pallas_call(
        body,
        grid=(R // blk_rows,),
        in_specs=[
            pl.BlockSpec((blk_rows, D), lambda i: (i, 0)),
            pl.BlockSpec((D, D), lambda i: (0, 0)),
            pl.BlockSpec((D, D), lambda i: (0, 0)),
            pl.BlockSpec((1, D), lambda i: (0, 0)),
        ],
        out_specs=[
            pl.BlockSpec((blk_rows, D), lambda i: (i, 0)),
            pl.BlockSpec((blk_rows, D), lambda i: (i, 0)),
        ],
        out_shape=[
            jax.ShapeDtypeStruct((R, D), jnp.float32),
            jax.ShapeDtypeStruct((R, D), jnp.float32),
        ],
    )(h, W_rel, W_root, b)


def _fused_dense_two(parts, r_prev, W_rel, W_root, b, blk_rows):
    _, R, D = parts.shape

    def body(p_ref, rp_ref, wr_ref, wo_ref, b_ref, m_ref, r_ref):
        hb = jnp.maximum(p_ref[0] + p_ref[1] + rp_ref[...], 0.0)
        dn = (((1,), (1,)), ((), ()))
        m_ref[...] = lax.dot_general(hb, wr_ref[...], dn,
                                     preferred_element_type=jnp.float32)
        r_ref[...] = lax.dot_general(hb, wo_ref[...], dn,
                                     preferred_element_type=jnp.float32) + b_ref[...]

    return pl.pallas_call(
        body,
        grid=(R // blk_rows,),
        in_specs=[
            pl.BlockSpec((2, blk_rows, D), lambda i: (0, i, 0)),
            pl.BlockSpec((blk_rows, D), lambda i: (i, 0)),
            pl.BlockSpec((D, D), lambda i: (0, 0)),
            pl.BlockSpec((D, D), lambda i: (0, 0)),
            pl.BlockSpec((1, D), lambda i: (0, 0)),
        ],
        out_specs=[
            pl.BlockSpec((blk_rows, D), lambda i: (i, 0)),
            pl.BlockSpec((blk_rows, D), lambda i: (i, 0)),
        ],
        out_shape=[
            jax.ShapeDtypeStruct((R, D), jnp.float32),
            jax.ShapeDtypeStruct((R, D), jnp.float32),
        ],
    )(parts, r_prev, W_rel, W_root, b)


def _combine(parts, r, blk_rows):
    _, R, D = parts.shape

    def body(p_ref, r_ref, o_ref):
        o_ref[...] = jnp.maximum(p_ref[0] + p_ref[1] + r_ref[...], 0.0)

    return pl.pallas_call(
        body,
        grid=(R // blk_rows,),
        in_specs=[
            pl.BlockSpec((2, blk_rows, D), lambda i: (0, i, 0)),
            pl.BlockSpec((blk_rows, D), lambda i: (i, 0)),
        ],
        out_specs=pl.BlockSpec((blk_rows, D), lambda i: (i, 0)),
        out_shape=jax.ShapeDtypeStruct((R, D), jnp.float32),
    )(parts, r)


def _sc_segsum(m, sdb, zeros, n_acc, rows_per_tile, blocks_per_tile):
    D = m.shape[1]
    bpt = blocks_per_tile
    mesh = plsc.VectorSubcoreMesh(core_axis_name="c", subcore_axis_name="s",
                                  num_cores=_NUM_CORES,
                                  num_subcores=_NUM_SUBCORES)

    @functools.partial(
        pl.kernel,
        out_type=jax.ShapeDtypeStruct((_NUM_CORES, n_acc, D), jnp.float32),
        mesh=mesh,
        scratch_types=[
            pltpu.VMEM((2, _BLK_EDGES), jnp.int32),
            pltpu.VMEM((_BLK_EDGES, D), jnp.float32),
            pltpu.VMEM_SHARED((n_acc, D), jnp.float32),
            pltpu.SemaphoreType.DMA,
        ],
    )
    def k(m_hbm, sdb_hbm, z_hbm, out_hbm, idx, rows, acc_sh, gsem):
        c = lax.axis_index("c")
        s = lax.axis_index("s")
        my_rows = pl.ds(s * rows_per_tile, rows_per_tile)
        base = (c * _NUM_SUBCORES + s) * bpt
        pltpu.sync_copy(z_hbm, acc_sh.at[my_rows])
        plsc.subcore_barrier()

        @pl.loop(0, bpt)
        def _(j):
            pltpu.sync_copy(sdb_hbm.at[base + j], idx)
            pltpu.async_copy(m_hbm.at[idx.at[0]], rows, gsem).wait()
            pltpu.sync_copy(rows, acc_sh.at[idx.at[1]], add=True)

        plsc.subcore_barrier()
        pltpu.sync_copy(acc_sh.at[my_rows], out_hbm.at[c].at[my_rows])

    return k(m, sdb, zeros)


def kernel(x, edge_index, W1_rel, W1_root, b1, W2_rel, W2_root, b2):
    N, D = x.shape
    E = edge_index.shape[1]
    nw = _NUM_CORES * _NUM_SUBCORES

    blocks_per_tile = -(-E // (nw * _BLK_EDGES))
    e_pad = nw * _BLK_EDGES * blocks_per_tile
    rows_per_tile = _round_up(-(-(N + 1) // _NUM_SUBCORES), 64)
    n_acc = _NUM_SUBCORES * rows_per_tile

    src = edge_index[0].astype(jnp.int32)
    dst = edge_index[1].astype(jnp.int32)
    pad = e_pad - E
    nblocks = e_pad // _BLK_EDGES
    srcb = jnp.pad(src, (0, pad), constant_values=N).reshape(nblocks,
                                                             _BLK_EDGES)
    dstb = jnp.pad(dst, (0, pad), constant_values=N).reshape(nblocks,
                                                             _BLK_EDGES)
    sdb = jnp.stack([srcb, dstb], axis=1)
    xp = jnp.pad(x, ((0, n_acc - N), (0, 0)))
    zeros = jnp.zeros((rows_per_tile, D), jnp.float32)
    b1r = b1.reshape(1, D)
    b2r = b2.reshape(1, D)

    blk_rows = 1024 if n_acc % 1024 == 0 else 64

    m1, r1 = _dense_two(xp, W1_rel, W1_root, b1r, blk_rows)
    parts1 = _sc_segsum(m1, sdb, zeros, n_acc, rows_per_tile, blocks_per_tile)
    m2, r2 = _fused_dense_two(parts1, r1, W2_rel, W2_root, b2r, blk_rows)
    parts2 = _sc_segsum(m2, sdb, zeros, n_acc, rows_per_tile, blocks_per_tile)
    out = _combine(parts2, r2, blk_rows)
    return out[:N]
